# baseline (speedup 1.0000x reference)
import functools

import jax
import jax.numpy as jnp
from jax import lax
from jax.experimental import pallas as pl
from jax.experimental.pallas import tpu as pltpu
from jax.experimental.pallas import tpu_sc as plsc

BATCH = 16384
L = 16
EMB_ROWS = (1559, 16, 5, 3, 4, 10, 4)
EMB_DIMS = (50, 8, 3, 2, 3, 5, 3)
NCH = tuple(-(-d // L) for d in EMB_DIMS)
N_CONT = 15
CHUNKS = sum(NCH)
XW = CHUNKS * L
R_TBL = sum(n * c for n, c in zip(EMB_ROWS, NCH))
IDX_MINOR = 128

_BASES = []
_b = 0
for _n, _c in zip(EMB_ROWS, NCH):
    _BASES.append(_b)
    _b += _n * _c
_POS = []
for _f in range(len(EMB_ROWS)):
    for _j in range(NCH[_f]):
        _POS.append((_f, _BASES[_f] + _j, NCH[_f]))


def _sc_gather(src, xe_t, off, nb):
    info = plsc.get_sparse_core_info()
    nw = info.num_cores * info.num_subcores
    per_w = nb // nw
    n_k = per_w // IDX_MINOR
    nf = len(EMB_ROWS)
    mesh = plsc.VectorSubcoreMesh(core_axis_name="c", subcore_axis_name="s")

    @functools.partial(
        pl.kernel,
        mesh=mesh,
        compiler_params=pltpu.CompilerParams(use_tc_tiling_on_sc=False),
        out_type=jax.ShapeDtypeStruct((nb, XW), jnp.float32),
        scratch_types=[
            pltpu.VMEM((nf, per_w), jnp.int32),
            pltpu.VMEM((CHUNKS * n_k, IDX_MINOR), jnp.int32),
            pltpu.VMEM((CHUNKS * per_w, L), jnp.float32),
            pltpu.VMEM_SHARED((R_TBL, L), jnp.float32),
            pltpu.SemaphoreType.DMA,
            pltpu.SemaphoreType.DMA,
        ],
    )
    def k(src_hbm, xe_hbm, out_hbm, xe_v, idx_v, rows_v, spt, sem, sem2):
        sub = lax.axis_index("s")
        wid = sub * info.num_cores + lax.axis_index("c")
        b0 = wid * per_w

        @pl.when(sub == 0)
        def _stage():
            pltpu.sync_copy(src_hbm, spt)

        pltpu.sync_copy(xe_hbm.at[:, pl.ds(off + b0, per_w)], xe_v)

        for p, (f, cbase, nch) in enumerate(_POS):
            for g in range(per_w // L):
                xv = xe_v[f, pl.ds(g * L, L)]
                idx_v[p * n_k + g // 8, pl.ds((g % 8) * L, L)] = (
                    xv * nch + cbase)

        plsc.subcore_barrier()

        for j in range(CHUNKS * n_k):
            pltpu.async_copy(
                spt.at[idx_v.at[j]],
                rows_v.at[pl.ds(j * IDX_MINOR, IDX_MINOR)],
                sem,
            )
        pltpu.make_async_copy(
            src_hbm.at[pl.ds(0, CHUNKS * per_w)], rows_v, sem
        ).wait()
        for p in range(CHUNKS):
            pltpu.async_copy(
                rows_v.at[pl.ds(p * per_w, per_w)],
                out_hbm.at[pl.ds(b0, per_w), pl.ds(p * L, L)],
                sem2,
            )
        for p in range(CHUNKS):
            pltpu.make_async_copy(
                src_hbm.at[pl.ds(0, per_w)],
                rows_v.at[pl.ds(p * per_w, per_w)],
                sem2,
            ).wait()

    return k(src, xe_t)


def _mlp_body(x_ref, xc_ref, w0_ref, wc_ref, b0_ref, g0_ref, be0_ref,
              rm0_ref, rv0_ref,
              w1_ref, b1_ref, g1_ref, be1_ref, rm1_ref, rv1_ref,
              w2_ref, b2_ref, g2_ref, be2_ref, rm2_ref, rv2_ref,
              w3_ref, b3_ref, out_ref):
    dn = (((1,), (1,)), ((), ()))
    h = lax.dot_general(x_ref[...], w0_ref[...], dn,
                        preferred_element_type=jnp.float32)
    h = h + lax.dot_general(xc_ref[...], wc_ref[...], dn,
                            preferred_element_type=jnp.float32)
    first = True
    for w_ref, b_ref, g_ref, be_ref, rm_ref, rv_ref in (
        (None, b0_ref, g0_ref, be0_ref, rm0_ref, rv0_ref),
        (w1_ref, b1_ref, g1_ref, be1_ref, rm1_ref, rv1_ref),
        (w2_ref, b2_ref, g2_ref, be2_ref, rm2_ref, rv2_ref),
    ):
        if not first:
            h = lax.dot_general(h, w_ref[...], dn,
                                preferred_element_type=jnp.float32)
        first = False
        y = h + b_ref[...][None, :]
        scale = (g_ref[...] * lax.rsqrt(rv_ref[...] + 1e-5))[None, :]
        y = (y - rm_ref[...][None, :]) * scale + be_ref[...][None, :]
        h = jnp.maximum(y, 0.0)
    o = lax.dot_general(h, w3_ref[...], dn, preferred_element_type=jnp.float32)
    out_ref[...] = o + b3_ref[0]


def _mlp(x, xc, w0e, w0c, b0, g0, be0, rm0, rv0, w1, b1, g1, be1, rm1, rv1,
         w2, b2, g2, be2, rm2, rv2, w3, b3, block_b=2048):
    nb = x.shape[0]
    grid = (nb // block_b,)

    def full2(shape):
        return pl.BlockSpec(shape, lambda i: (0, 0))

    def full1(shape):
        return pl.BlockSpec(shape, lambda i: (0,))

    h0, h1, h2 = w0e.shape[0], w1.shape[0], w2.shape[0]
    in_specs = [
        pl.BlockSpec((block_b, XW), lambda i: (i, 0)),
        pl.BlockSpec((block_b, N_CONT), lambda i: (i, 0)),
        full2(w0e.shape), full2(w0c.shape), full1((h0,)), full1((h0,)),
        full1((h0,)), full1((h0,)), full1((h0,)),
        full2(w1.shape), full1((h1,)), full1((h1,)), full1((h1,)),
        full1((h1,)), full1((h1,)),
        full2(w2.shape), full1((h2,)), full1((h2,)), full1((h2,)),
        full1((h2,)), full1((h2,)),
        full2(w3.shape),
        pl.BlockSpec(memory_space=pltpu.MemorySpace.SMEM),
    ]
    return pl.pallas_call(
        _mlp_body,
        grid=grid,
        in_specs=in_specs,
        out_specs=pl.BlockSpec((block_b, 8), lambda i: (i, 0)),
        out_shape=jax.ShapeDtypeStruct((nb, 8), jnp.float32),
    )(x, xc, w0e, w0c, b0, g0, be0, rm0, rv0, w1, b1, g1, be1, rm1, rv1,
      w2, b2, g2, be2, rm2, rv2, w3, b3)


def _pack_source(tables):
    parts = []
    for t, nch in zip(tables, NCH):
        n, d = t.shape
        tp = jnp.pad(t, ((0, 0), (0, nch * L - d)))
        parts.append(tp.reshape(n * nch, L))
    return jnp.concatenate(parts, axis=0)


def _pad_w0e(w0):
    segs = []
    off = 0
    for d, nch in zip(EMB_DIMS, NCH):
        segs.append(jnp.pad(w0[:, off:off + d], ((0, 0), (0, nch * L - d))))
        off += d
    return jnp.concatenate(segs, axis=1)


def kernel(x_emb, x_cont, emb0, emb1, emb2, emb3, emb4, emb5, emb6,
           W0, b0, g0, be0, rm0, rv0,
           W1, b1, g1, be1, rm1, rv1,
           W2, b2, g2, be2, rm2, rv2,
           W3, b3):
    tables = (emb0, emb1, emb2, emb3, emb4, emb5, emb6)
    src = _pack_source(tables)
    xe_t = x_emb.astype(jnp.int32).T
    w0e = _pad_w0e(W0)
    w0c = W0[:, sum(EMB_DIMS):]
    w3p = jnp.pad(W3, ((0, 7), (0, 0)))
    half = BATCH // 2
    outs = []
    for off in (0, half):
        x = _sc_gather(src, xe_t, off, half)
        outs.append(_mlp(x, x_cont[off:off + half], w0e, w0c,
                         b0, g0, be0, rm0, rv0,
                         W1, b1, g1, be1, rm1, rv1,
                         W2, b2, g2, be2, rm2, rv2, w3p, b3))
    return jnp.concatenate(outs, axis=0)[:, 0]


# speedup vs baseline: 1.0137x; 1.0137x over previous
import functools

import jax
import jax.numpy as jnp
from jax import lax
from jax.experimental import pallas as pl
from jax.experimental.pallas import tpu as pltpu
from jax.experimental.pallas import tpu_sc as plsc

BATCH = 16384
L = 16
EMB_ROWS = (1559, 16, 5, 3, 4, 10, 4)
EMB_DIMS = (50, 8, 3, 2, 3, 5, 3)
NCH = tuple(-(-d // L) for d in EMB_DIMS)
N_CONT = 15
CHUNKS = sum(NCH)
XW = CHUNKS * L
R_TBL = sum(n * c for n, c in zip(EMB_ROWS, NCH))
IDX_MINOR = 128

_BASES = []
_b = 0
for _n, _c in zip(EMB_ROWS, NCH):
    _BASES.append(_b)
    _b += _n * _c
_POS = []
for _f in range(len(EMB_ROWS)):
    for _j in range(NCH[_f]):
        _POS.append((_f, _BASES[_f] + _j, NCH[_f]))


def _sc_gather(src, xe_t):
    info = plsc.get_sparse_core_info()
    nw = info.num_cores * info.num_subcores
    per_w = BATCH // nw
    n_k = per_w // IDX_MINOR
    nf = len(EMB_ROWS)
    mesh = plsc.VectorSubcoreMesh(core_axis_name="c", subcore_axis_name="s")

    @functools.partial(
        pl.kernel,
        mesh=mesh,
        compiler_params=pltpu.CompilerParams(use_tc_tiling_on_sc=False),
        out_type=jax.ShapeDtypeStruct((BATCH, XW), jnp.float32),
        scratch_types=[
            pltpu.VMEM((nf, per_w), jnp.int32),
            pltpu.VMEM((CHUNKS * n_k, IDX_MINOR), jnp.int32),
            pltpu.VMEM((CHUNKS * per_w, L), jnp.float32),
            pltpu.VMEM_SHARED((R_TBL, L), jnp.float32),
            pltpu.SemaphoreType.DMA,
            pltpu.SemaphoreType.DMA,
        ],
    )
    def k(src_hbm, xe_hbm, out_hbm, xe_v, idx_v, rows_v, spt, sem, sem2):
        sub = lax.axis_index("s")
        wid = sub * info.num_cores + lax.axis_index("c")
        b0 = wid * per_w

        @pl.when(sub == 0)
        def _stage():
            pltpu.sync_copy(src_hbm, spt)

        pltpu.sync_copy(xe_hbm.at[:, pl.ds(b0, per_w)], xe_v)

        for p, (f, cbase, nch) in enumerate(_POS):
            for g in range(per_w // L):
                xv = xe_v[f, pl.ds(g * L, L)]
                idx_v[p * n_k + g // 8, pl.ds((g % 8) * L, L)] = (
                    xv * nch + cbase)

        plsc.subcore_barrier()

        for j in range(CHUNKS * n_k):
            pltpu.async_copy(
                spt.at[idx_v.at[j]],
                rows_v.at[pl.ds(j * IDX_MINOR, IDX_MINOR)],
                sem,
            )
        pltpu.make_async_copy(
            src_hbm.at[pl.ds(0, CHUNKS * per_w)], rows_v, sem
        ).wait()
        for p in range(CHUNKS):
            pltpu.async_copy(
                rows_v.at[pl.ds(p * per_w, per_w)],
                out_hbm.at[pl.ds(b0, per_w), pl.ds(p * L, L)],
                sem2,
            )
        for p in range(CHUNKS):
            pltpu.make_async_copy(
                src_hbm.at[pl.ds(0, per_w)],
                rows_v.at[pl.ds(p * per_w, per_w)],
                sem2,
            ).wait()

    return k(src, xe_t)


def _mlp_body(x_ref, xc_ref, w0_ref, wc_ref, b0_ref, g0_ref, be0_ref,
              rm0_ref, rv0_ref,
              w1_ref, b1_ref, g1_ref, be1_ref, rm1_ref, rv1_ref,
              w2_ref, b2_ref, g2_ref, be2_ref, rm2_ref, rv2_ref,
              w3_ref, b3_ref, out_ref):
    dn = (((1,), (1,)), ((), ()))
    h = lax.dot_general(x_ref[...], w0_ref[...], dn,
                        preferred_element_type=jnp.float32)
    h = h + lax.dot_general(xc_ref[...], wc_ref[...], dn,
                            preferred_element_type=jnp.float32)
    first = True
    for w_ref, b_ref, g_ref, be_ref, rm_ref, rv_ref in (
        (None, b0_ref, g0_ref, be0_ref, rm0_ref, rv0_ref),
        (w1_ref, b1_ref, g1_ref, be1_ref, rm1_ref, rv1_ref),
        (w2_ref, b2_ref, g2_ref, be2_ref, rm2_ref, rv2_ref),
    ):
        if not first:
            h = lax.dot_general(h, w_ref[...], dn,
                                preferred_element_type=jnp.float32)
        first = False
        y = h + b_ref[...][None, :]
        scale = (g_ref[...] * lax.rsqrt(rv_ref[...] + 1e-5))[None, :]
        y = (y - rm_ref[...][None, :]) * scale + be_ref[...][None, :]
        h = jnp.maximum(y, 0.0)
    o = lax.dot_general(h, w3_ref[...], dn, preferred_element_type=jnp.float32)
    out_ref[...] = o + b3_ref[0]


def _mlp(x, xc, w0e, w0c, b0, g0, be0, rm0, rv0, w1, b1, g1, be1, rm1, rv1,
         w2, b2, g2, be2, rm2, rv2, w3, b3, block_b=2048):
    grid = (BATCH // block_b,)

    def full2(shape):
        return pl.BlockSpec(shape, lambda i: (0, 0))

    def full1(shape):
        return pl.BlockSpec(shape, lambda i: (0,))

    h0, h1, h2 = w0e.shape[0], w1.shape[0], w2.shape[0]
    in_specs = [
        pl.BlockSpec((block_b, XW), lambda i: (i, 0)),
        pl.BlockSpec((block_b, N_CONT), lambda i: (i, 0)),
        full2(w0e.shape), full2(w0c.shape), full1((h0,)), full1((h0,)),
        full1((h0,)), full1((h0,)), full1((h0,)),
        full2(w1.shape), full1((h1,)), full1((h1,)), full1((h1,)),
        full1((h1,)), full1((h1,)),
        full2(w2.shape), full1((h2,)), full1((h2,)), full1((h2,)),
        full1((h2,)), full1((h2,)),
        full2(w3.shape),
        pl.BlockSpec(memory_space=pltpu.MemorySpace.SMEM),
    ]
    return pl.pallas_call(
        _mlp_body,
        grid=grid,
        in_specs=in_specs,
        out_specs=pl.BlockSpec((block_b, 8), lambda i: (i, 0)),
        out_shape=jax.ShapeDtypeStruct((BATCH, 8), jnp.float32),
    )(x, xc, w0e, w0c, b0, g0, be0, rm0, rv0, w1, b1, g1, be1, rm1, rv1,
      w2, b2, g2, be2, rm2, rv2, w3, b3)


def _pack_source(tables):
    parts = []
    for t, nch in zip(tables, NCH):
        n, d = t.shape
        tp = jnp.pad(t, ((0, 0), (0, nch * L - d)))
        parts.append(tp.reshape(n * nch, L))
    return jnp.concatenate(parts, axis=0)


def _pad_w0e(w0):
    segs = []
    off = 0
    for d, nch in zip(EMB_DIMS, NCH):
        segs.append(jnp.pad(w0[:, off:off + d], ((0, 0), (0, nch * L - d))))
        off += d
    return jnp.concatenate(segs, axis=1)


def kernel(x_emb, x_cont, emb0, emb1, emb2, emb3, emb4, emb5, emb6,
           W0, b0, g0, be0, rm0, rv0,
           W1, b1, g1, be1, rm1, rv1,
           W2, b2, g2, be2, rm2, rv2,
           W3, b3):
    tables = (emb0, emb1, emb2, emb3, emb4, emb5, emb6)
    src = _pack_source(tables)
    x = _sc_gather(src, x_emb.astype(jnp.int32).T)
    w0e = _pad_w0e(W0)
    w0c = W0[:, sum(EMB_DIMS):]
    w3p = jnp.pad(W3, ((0, 7), (0, 0)))
    out = _mlp(x, x_cont, w0e, w0c, b0, g0, be0, rm0, rv0,
               W1, b1, g1, be1, rm1, rv1,
               W2, b2, g2, be2, rm2, rv2, w3p, b3)
    return out[:, 0]
